# baseline (speedup 1.0000x reference)
import functools

import jax
import jax.numpy as jnp
import numpy as np
from jax import lax
from jax.experimental import pallas as pl
from jax.experimental.pallas import tpu as pltpu
from jax.experimental.pallas import tpu_sc as plsc

N_FIELDS = 26
VOCAB = 100000
EMB = 32
EPS = 1e-5

NC = 2
NS = 16
LANES = 16
NW = NC * NS

CHUNK = 128

VOCAB_PAD = 100096
FSTRIDE = VOCAB_PAD
RELAY_W = 4352
RELAY_J = VOCAB_PAD // RELAY_W
RELAY_R = RELAY_W // 4


def _compact_body(in_ref, out_ref):
    out_ref[...] = in_ref[0].T


def _tc_compact(tab_t):
    return pl.pallas_call(
        _compact_body,
        grid=(N_FIELDS, RELAY_J),
        in_specs=[pl.BlockSpec((1, EMB, RELAY_W), lambda f, j: (f, 0, j))],
        out_specs=pl.BlockSpec((RELAY_W, EMB), lambda f, j: (f * RELAY_J + j, 0)),
        out_shape=jax.ShapeDtypeStruct(
            (N_FIELDS * VOCAB_PAD, EMB), jnp.float32),
    )(tab_t)


def _gather_body(xcat_hbm, table_hbm, out_hbm, idx_v, rows_v, sem):
    tot = xcat_hbm.shape[0]
    per_w = tot // NW
    n_chunks = per_w // CHUNK
    wid = lax.axis_index("s") * NC + lax.axis_index("c")
    base_w = wid * per_w

    lane = jnp.arange(LANES, dtype=jnp.int32)

    def chunk(c, carry):
        base = base_w + c * CHUNK
        pltpu.sync_copy(xcat_hbm.at[pl.ds(base, CHUNK)], idx_v)
        for j in range(CHUNK // LANES):
            pos = base + j * LANES + lane
            f = lax.rem(pos, N_FIELDS)
            idx_v[pl.ds(j * LANES, LANES)] = (
                idx_v[pl.ds(j * LANES, LANES)] + f * FSTRIDE
            )
        pltpu.async_copy(table_hbm.at[idx_v], rows_v, sem).wait()
        pltpu.sync_copy(rows_v, out_hbm.at[pl.ds(base, CHUNK)])
        return carry

    lax.fori_loop(0, n_chunks, chunk, 0)


def _sc_gather(xcat_flat, table2d):
    tot = xcat_flat.shape[0]
    mesh = plsc.VectorSubcoreMesh(core_axis_name="c", subcore_axis_name="s")
    return pl.kernel(
        _gather_body,
        mesh=mesh,
        compiler_params=pltpu.CompilerParams(use_tc_tiling_on_sc=False),
        out_type=jax.ShapeDtypeStruct((tot, EMB), jnp.float32),
        scratch_types=[
            pltpu.VMEM((CHUNK,), jnp.int32),
            pltpu.VMEM((CHUNK, EMB), jnp.float32),
            pltpu.SemaphoreType.DMA,
        ],
    )(xcat_flat, table2d)


def _mlp_body(cat_ref, num_ref, w0a_ref, w0b_ref, s0_ref, t0_ref,
              w1_ref, s1_ref, t1_ref, w2_ref, s2_ref, t2_ref,
              wo_ref, bo_ref, out_ref):
    h = jnp.dot(cat_ref[...], w0a_ref[...], preferred_element_type=jnp.float32)
    h += jnp.dot(num_ref[...], w0b_ref[...], preferred_element_type=jnp.float32)
    h = jnp.maximum(h * s0_ref[...] + t0_ref[...], 0.0)
    h = jnp.dot(h, w1_ref[...], preferred_element_type=jnp.float32)
    h = jnp.maximum(h * s1_ref[...] + t1_ref[...], 0.0)
    h = jnp.dot(h, w2_ref[...], preferred_element_type=jnp.float32)
    h = jnp.maximum(h * s2_ref[...] + t2_ref[...], 0.0)
    out_ref[...] = (
        jnp.dot(h, wo_ref[...], preferred_element_type=jnp.float32) + bo_ref[...]
    )


def _tc_mlp(cat, xnum, w0a, w0b, s0, t0, w1, s1, t1, w2, s2, t2, wo, bo, blk):
    b = cat.shape[0]
    d_cat = cat.shape[1]
    d_num = xnum.shape[1]
    grid = (b // blk,)

    def full(a):
        return pl.BlockSpec(a.shape, lambda i: (0,) * a.ndim)

    return pl.pallas_call(
        _mlp_body,
        grid=grid,
        in_specs=[
            pl.BlockSpec((blk, d_cat), lambda i: (i, 0)),
            pl.BlockSpec((blk, d_num), lambda i: (i, 0)),
            full(w0a), full(w0b), full(s0), full(t0),
            full(w1), full(s1), full(t1),
            full(w2), full(s2), full(t2),
            full(wo), full(bo),
        ],
        out_specs=pl.BlockSpec((blk, 1), lambda i: (i, 0)),
        out_shape=jax.ShapeDtypeStruct((b, 1), jnp.float32),
    )(cat, xnum, w0a, w0b, s0, t0, w1, s1, t1, w2, s2, t2, wo, bo)


def kernel(X_cat, X_num, tables, W0, b0, g0, be0, W1, b1, g1, be1,
           W2, b2, g2, be2, W_out, b_out):
    b = X_cat.shape[0]
    d_cat = N_FIELDS * EMB

    tab_t = jnp.transpose(tables, (0, 2, 1))
    table2d = _tc_compact(tab_t)
    xcat_flat = X_cat.reshape(-1).astype(jnp.int32)
    cat = _sc_gather(xcat_flat, table2d).reshape(b, d_cat)

    k = np.float32(1.0) / jnp.sqrt(jnp.float32(1.0) + jnp.float32(EPS))
    s0 = (g0 * k).reshape(1, -1)
    t0 = (b0 * g0 * k + be0).reshape(1, -1)
    s1 = (g1 * k).reshape(1, -1)
    t1 = (b1 * g1 * k + be1).reshape(1, -1)
    s2 = (g2 * k).reshape(1, -1)
    t2 = (b2 * g2 * k + be2).reshape(1, -1)

    out = _tc_mlp(
        cat, X_num,
        W0[:d_cat], W0[d_cat:], s0, t0,
        W1, s1, t1, W2, s2, t2,
        W_out, b_out.reshape(1, -1),
        blk=1024,
    )
    return out


# speedup vs baseline: 1.3390x; 1.3390x over previous
import functools

import jax
import jax.numpy as jnp
import numpy as np
from jax import lax
from jax.experimental import pallas as pl
from jax.experimental.pallas import tpu as pltpu
from jax.experimental.pallas import tpu_sc as plsc

N_FIELDS = 26
VOCAB = 100000
EMB = 32
EPS = 1e-5

NC = 2
NS = 16
LANES = 16
NW = NC * NS

CHUNK = 128


def _gather_body(xcat_hbm, table_hbm, out_hbm, idx_v, rows_v, sem):
    tot = xcat_hbm.shape[0]
    per_w = tot // NW
    n_chunks = per_w // CHUNK
    wid = lax.axis_index("s") * NC + lax.axis_index("c")
    base_w = wid * per_w

    lane = jnp.arange(LANES, dtype=jnp.int32)

    def chunk(c, carry):
        base = base_w + c * CHUNK
        pltpu.sync_copy(xcat_hbm.at[pl.ds(base, CHUNK)], idx_v)
        for j in range(CHUNK // LANES):
            pos = base + j * LANES + lane
            f = lax.rem(pos, N_FIELDS)
            idx_v[pl.ds(j * LANES, LANES)] = (
                idx_v[pl.ds(j * LANES, LANES)] + f * VOCAB
            )
        pltpu.async_copy(table_hbm.at[idx_v], rows_v, sem).wait()
        pltpu.sync_copy(rows_v, out_hbm.at[pl.ds(base, CHUNK)])
        return carry

    lax.fori_loop(0, n_chunks, chunk, 0)


def _sc_gather(xcat_flat, table2d):
    tot = xcat_flat.shape[0]
    mesh = plsc.VectorSubcoreMesh(core_axis_name="c", subcore_axis_name="s")
    return pl.kernel(
        _gather_body,
        mesh=mesh,
        compiler_params=pltpu.CompilerParams(use_tc_tiling_on_sc=False),
        out_type=jax.ShapeDtypeStruct((tot, EMB), jnp.float32),
        scratch_types=[
            pltpu.VMEM((CHUNK,), jnp.int32),
            pltpu.VMEM((CHUNK, EMB), jnp.float32),
            pltpu.SemaphoreType.DMA,
        ],
    )(xcat_flat, table2d)


def _mlp_body(cat_ref, num_ref, w0a_ref, w0b_ref, s0_ref, t0_ref,
              w1_ref, s1_ref, t1_ref, w2_ref, s2_ref, t2_ref,
              wo_ref, bo_ref, out_ref):
    h = jnp.dot(cat_ref[...], w0a_ref[...], preferred_element_type=jnp.float32)
    h += jnp.dot(num_ref[...], w0b_ref[...], preferred_element_type=jnp.float32)
    h = jnp.maximum(h * s0_ref[...] + t0_ref[...], 0.0)
    h = jnp.dot(h, w1_ref[...], preferred_element_type=jnp.float32)
    h = jnp.maximum(h * s1_ref[...] + t1_ref[...], 0.0)
    h = jnp.dot(h, w2_ref[...], preferred_element_type=jnp.float32)
    h = jnp.maximum(h * s2_ref[...] + t2_ref[...], 0.0)
    out_ref[...] = (
        jnp.dot(h, wo_ref[...], preferred_element_type=jnp.float32) + bo_ref[...]
    )


def _tc_mlp(cat, xnum, w0a, w0b, s0, t0, w1, s1, t1, w2, s2, t2, wo, bo, blk):
    b = cat.shape[0]
    d_cat = cat.shape[1]
    d_num = xnum.shape[1]
    grid = (b // blk,)

    def full(a):
        return pl.BlockSpec(a.shape, lambda i: (0,) * a.ndim)

    return pl.pallas_call(
        _mlp_body,
        grid=grid,
        in_specs=[
            pl.BlockSpec((blk, d_cat), lambda i: (i, 0)),
            pl.BlockSpec((blk, d_num), lambda i: (i, 0)),
            full(w0a), full(w0b), full(s0), full(t0),
            full(w1), full(s1), full(t1),
            full(w2), full(s2), full(t2),
            full(wo), full(bo),
        ],
        out_specs=pl.BlockSpec((blk, 1), lambda i: (i, 0)),
        out_shape=jax.ShapeDtypeStruct((b, 1), jnp.float32),
    )(cat, xnum, w0a, w0b, s0, t0, w1, s1, t1, w2, s2, t2, wo, bo)


def kernel(X_cat, X_num, tables, W0, b0, g0, be0, W1, b1, g1, be1,
           W2, b2, g2, be2, W_out, b_out):
    b = X_cat.shape[0]
    d_cat = N_FIELDS * EMB

    table2d = tables.reshape(N_FIELDS * VOCAB, EMB)
    xcat_flat = X_cat.reshape(-1).astype(jnp.int32)
    cat = _sc_gather(xcat_flat, table2d).reshape(b, d_cat)

    k = np.float32(1.0) / jnp.sqrt(jnp.float32(1.0) + jnp.float32(EPS))
    s0 = (g0 * k).reshape(1, -1)
    t0 = (b0 * g0 * k + be0).reshape(1, -1)
    s1 = (g1 * k).reshape(1, -1)
    t1 = (b1 * g1 * k + be1).reshape(1, -1)
    s2 = (g2 * k).reshape(1, -1)
    t2 = (b2 * g2 * k + be2).reshape(1, -1)

    out = _tc_mlp(
        cat, X_num,
        W0[:d_cat], W0[d_cat:], s0, t0,
        W1, s1, t1, W2, s2, t2,
        W_out, b_out.reshape(1, -1),
        blk=1024,
    )
    return out
